# baseline (speedup 1.0000x reference)
import jax
import jax.numpy as jnp
from jax.experimental import pallas as pl
from jax.experimental.pallas import tpu as pltpu


def _copy_body(in_ref, out_ref):
    out_ref[...] = in_ref[...]


def _copy_1d(x, block):
    n = x.shape[0]
    grid = n // block
    return pl.pallas_call(
        _copy_body,
        out_shape=jax.ShapeDtypeStruct(x.shape, x.dtype),
        grid=(grid,),
        in_specs=[pl.BlockSpec((block,), lambda i: (i,))],
        out_specs=pl.BlockSpec((block,), lambda i: (i,)),
    )(x)


def kernel(emb_user, emb_item):
    u_shape, i_shape = emb_user.shape, emb_item.shape
    u1 = emb_user.reshape(-1)
    i1 = emb_item.reshape(-1)
    out_u = _copy_1d(u1, 2560000)
    out_i = _copy_1d(i1, 1280000)
    return (out_u.reshape(u_shape), out_i.reshape(i_shape))


# speedup vs baseline: 1.2767x; 1.2767x over previous
import jax
import jax.numpy as jnp
from jax import lax
from jax.experimental import pallas as pl
from jax.experimental.pallas import tpu as pltpu
from jax.experimental.pallas import tpu_sc as plsc

_B = 400
_NW = 32


def _sc_copy_body(u_in, u_out, bufs, in_sems, out_sems):
    wid = lax.axis_index("c") * 16 + lax.axis_index("s")

    def phase(src, dst, n_chunks):
        iters = (n_chunks + _NW - 1) // _NW

        def masked(j, fn):
            c = wid + _NW * j

            @pl.when(c < n_chunks)
            def _():
                fn(c)

        def in_copy(j, c):
            return pltpu.make_async_copy(
                src.at[pl.ds(c * _B, _B)], bufs.at[j % 2], in_sems.at[j % 2])

        def out_copy(j, c):
            return pltpu.make_async_copy(
                bufs.at[j % 2], dst.at[pl.ds(c * _B, _B)], out_sems.at[j % 2])

        masked(0, lambda c: in_copy(0, c).start())
        for j in range(iters):
            masked(j, lambda c, j=j: in_copy(j, c).wait())
            masked(j, lambda c, j=j: out_copy(j, c).start())
            if j + 1 < iters:
                if j >= 1:
                    masked(j - 1, lambda c, j=j: out_copy(j - 1, c).wait())
                masked(j + 1, lambda c, j=j: in_copy(j + 1, c).start())
        if iters >= 2:
            masked(iters - 2, lambda c: out_copy(iters - 2, c).wait())
        if iters:
            masked(iters - 1, lambda c: out_copy(iters - 1, c).wait())

    phase(u_in, u_out, u_in.shape[0] // _B)


def _sc_copy(x):
    mesh = plsc.VectorSubcoreMesh(core_axis_name="c", subcore_axis_name="s")
    run = pl.kernel(
        _sc_copy_body,
        out_type=jax.ShapeDtypeStruct(x.shape, x.dtype),
        mesh=mesh,
        scratch_types=[
            pltpu.VMEM((2, _B, 64), jnp.float32),
            pltpu.SemaphoreType.DMA((2,)),
            pltpu.SemaphoreType.DMA((2,)),
        ],
    )
    return run(x)


def _tc_copy_body(in_ref, out_ref):
    out_ref[...] = in_ref[...]


def _tc_copy(x, block_rows):
    n_rows, width = x.shape
    grid = n_rows // block_rows
    return pl.pallas_call(
        _tc_copy_body,
        out_shape=jax.ShapeDtypeStruct(x.shape, x.dtype),
        grid=(grid,),
        in_specs=[pl.BlockSpec((block_rows, width), lambda i: (i, 0))],
        out_specs=pl.BlockSpec((block_rows, width), lambda i: (i, 0)),
    )(x)


def kernel(emb_user, emb_item):
    out_user = _sc_copy(emb_user)
    out_item = _tc_copy(emb_item, 10000)
    return (out_user, out_item)


# speedup vs baseline: 1.3024x; 1.0201x over previous
import jax
import jax.numpy as jnp
from jax.experimental import pallas as pl
from jax.experimental.pallas import tpu as pltpu


def _copy_body(in_ref, out_ref):
    out_ref[...] = in_ref[...]


def _copy(x, block_rows):
    n_rows, width = x.shape
    grid = n_rows // block_rows
    return pl.pallas_call(
        _copy_body,
        out_shape=jax.ShapeDtypeStruct(x.shape, x.dtype),
        grid=(grid,),
        in_specs=[pl.BlockSpec((block_rows, width), lambda i: (i, 0))],
        out_specs=pl.BlockSpec((block_rows, width), lambda i: (i, 0)),
        compiler_params=pltpu.CompilerParams(
            needs_layout_passes=False,
        ),
    )(x)


def kernel(emb_user, emb_item):
    out_user = _copy(emb_user, 8000)
    out_item = _copy(emb_item, 10000)
    return (out_user, out_item)


# speedup vs baseline: 1.7509x; 1.3444x over previous
import jax
import jax.numpy as jnp
from jax.experimental import pallas as pl
from jax.experimental.pallas import tpu as pltpu

_T = 16


def _copy_body(in_ref, out_ref):
    out_ref[...] = in_ref[...]


def _copy3d(x, block):
    n, t, width = x.shape
    grid = n // block
    return pl.pallas_call(
        _copy_body,
        out_shape=jax.ShapeDtypeStruct(x.shape, x.dtype),
        grid=(grid,),
        in_specs=[pl.BlockSpec((block, t, width), lambda i: (i, 0, 0))],
        out_specs=pl.BlockSpec((block, t, width), lambda i: (i, 0, 0)),
    )(x)


def kernel(emb_user, emb_item):
    u_shape, i_shape = emb_user.shape, emb_item.shape
    u3 = emb_user.reshape(-1, _T, 64)
    i3 = emb_item.reshape(-1, _T, 64)
    out_u = _copy3d(u3, 500)
    out_i = _copy3d(i3, 625)
    return (out_u.reshape(u_shape), out_i.reshape(i_shape))
